# baseline (speedup 1.0000x reference)
import functools
import jax
import jax.numpy as jnp
from jax import lax
from jax.experimental import pallas as pl
from jax.experimental.pallas import tpu as pltpu

_N_HASHES = 4
_CHANNELS = 64
_REDUCTION = 4
_CHUNK = 144
_CR = _CHANNELS // _REDUCTION


def _attn_body(x_c, x_p, x_n, y_c, y_p, y_n, f_c, f_p, f_n,
               fc1_w, fc1_b, fc2_w, fc2_b, out_ref):
    xq = x_c[0, 0]

    def nrm(t):
        t = t[0, 0]
        n = jnp.sqrt(jnp.sum(t * t, axis=-1, keepdims=True))
        return t / jnp.maximum(n, 5e-05)

    xm = jnp.concatenate([nrm(x_c), nrm(x_p), nrm(x_n)], axis=0)
    yc = jnp.concatenate([y_c[0, 0], y_p[0, 0], y_n[0, 0]], axis=0)
    fc = jnp.concatenate([f_c[0, 0], f_p[0, 0], f_n[0, 0]], axis=0)

    h1 = jax.nn.relu(
        lax.dot_general(fc, fc1_w[...], (((1,), (1,)), ((), ())),
                        preferred_element_type=jnp.float32) + fc1_b[...])
    g = lax.dot_general(fc2_w[...], h1, (((1,), (1,)), ((), ())),
                        preferred_element_type=jnp.float32)
    qk = lax.dot_general(xq, xm, (((1,), (1,)), ((), ())),
                         preferred_element_type=jnp.float32)
    raw = qk + g + fc2_b[0][:, None]
    m = jnp.max(raw, axis=-1, keepdims=True)
    e = jnp.exp(raw - m)
    s = jnp.sum(e, axis=-1, keepdims=True)
    ret = lax.dot_general(e, yc, (((1,), (0,)), ((), ())),
                          preferred_element_type=jnp.float32) / s
    bscore = m + jnp.log(s)
    out_ref[0, 0] = jnp.concatenate([ret, bscore], axis=-1)


def _bucket_attention(x_b, y_b, f_b, fc1_w, fc1_b, fc2_w, fc2_b):
    G, K = x_b.shape[0], x_b.shape[1]

    def cur(h, k):
        return (h, k, 0, 0)

    def prv(h, k):
        return (h, (k - 1) % K, 0, 0)

    def nxt(h, k):
        return (h, (k + 1) % K, 0, 0)

    def spec(c, im):
        return pl.BlockSpec((1, 1, _CHUNK, c), im)

    full2 = lambda shape: pl.BlockSpec(shape, lambda h, k: (0, 0))

    return pl.pallas_call(
        _attn_body,
        grid=(G, K),
        in_specs=[
            spec(_CR, cur), spec(_CR, prv), spec(_CR, nxt),
            spec(_CHANNELS, cur), spec(_CHANNELS, prv), spec(_CHANNELS, nxt),
            spec(_CHANNELS, cur), spec(_CHANNELS, prv), spec(_CHANNELS, nxt),
            full2((_CHUNK, _CHANNELS)), full2((1, _CHUNK)),
            full2((_CHUNK, _CHUNK)), full2((1, _CHUNK)),
        ],
        out_specs=spec(_CHANNELS + 1, cur),
        out_shape=jax.ShapeDtypeStruct((G, K, _CHUNK, _CHANNELS + 1), jnp.float32),
    )(x_b, x_b, x_b, y_b, y_b, y_b, f_b, f_b, f_b,
      fc1_w, fc1_b.reshape(1, -1), fc2_w, fc2_b.reshape(1, -1))


def _conv2d_relu(x, w, b):
    y = lax.conv_general_dilated(x, w, window_strides=(1, 1), padding='SAME',
                                 dimension_numbers=('NCHW', 'OIHW', 'NCHW'))
    return jax.nn.relu(y + b[None, :, None, None])


def kernel(input, cm_w, cm_b, ca_w, ca_b, cf_w, cf_b, fc1_w, fc1_b, fc2_w, fc2_b, random_rotations):
    x_nchw = jnp.transpose(input, (0, 3, 1, 2))
    N, _, H, W = x_nchw.shape
    L = H * W
    x_embed = _conv2d_relu(x_nchw, cm_w, cm_b).reshape(N, -1, L).transpose(0, 2, 1)
    y_embed = _conv2d_relu(x_nchw, ca_w, ca_b).reshape(N, -1, L).transpose(0, 2, 1)
    fc_embed = _conv2d_relu(x_nchw, cf_w, cf_b).reshape(N, -1, L).transpose(0, 2, 1)
    C = x_embed.shape[-1]
    hb = min(L // _CHUNK + (L // _CHUNK) % 2, 128)

    rot = random_rotations.reshape(C, _N_HASHES, hb)
    rotated = jnp.einsum('btf,fhi->bhti', x_embed, rot)
    hash_codes = jnp.argmax(rotated, axis=-1)
    offsets = (jnp.arange(_N_HASHES) * hb).reshape(1, -1, 1)
    hash_codes = (hash_codes + offsets).reshape(N, -1)
    indices = jnp.argsort(hash_codes, axis=-1)
    undo_sort = jnp.argsort(indices, axis=-1)
    mod_indices = indices % L
    bidx = jnp.arange(N)[:, None]
    x_s = x_embed[bidx, mod_indices]
    y_s = y_embed[bidx, mod_indices]
    f_s = fc_embed[bidx, mod_indices]

    K = L // _CHUNK
    G = N * _N_HASHES
    x_b = x_s.reshape(G, K, _CHUNK, C)
    y_b = y_s.reshape(G, K, _CHUNK, _CHANNELS)
    f_b = f_s.reshape(G, K, _CHUNK, _CHANNELS)

    out65 = _bucket_attention(x_b, y_b, f_b, fc1_w, fc1_b, fc2_w, fc2_b)

    ret = out65[..., :_CHANNELS].reshape(N, _N_HASHES * L, _CHANNELS)
    bscore = out65[..., _CHANNELS].reshape(N, _N_HASHES * L)
    ret = ret[bidx, undo_sort]
    bscore = jnp.take_along_axis(bscore, undo_sort, axis=1)
    ret = ret.reshape(N, _N_HASHES, L, _CHANNELS)
    bscore = bscore.reshape(N, _N_HASHES, L, 1)
    probs = jax.nn.softmax(bscore, axis=1)
    ret = jnp.sum(ret * probs, axis=1).reshape(N, H, W, -1)
    return ret + input


# speedup vs baseline: 5.1977x; 5.1977x over previous
import functools
import jax
import jax.numpy as jnp
from jax import lax
from jax.experimental import pallas as pl
from jax.experimental.pallas import tpu as pltpu
from jax.experimental.pallas import tpu_sc as plsc

_N_HASHES = 4
_CHANNELS = 64
_REDUCTION = 4
_CHUNK = 144
_CR = _CHANNELS // _REDUCTION
_ROW = 256
_OROW = 128

_NW = 32
_SCCHUNK = 128


def _embed_body(x_ref, y_ref, f_ref, fc1_w, fc1_b, fc2_w, out_ref):
    h1 = jax.nn.relu(
        lax.dot_general(f_ref[...], fc1_w[...], (((1,), (1,)), ((), ())),
                        preferred_element_type=jnp.float32) + fc1_b[...])
    t = lax.dot_general(h1, fc2_w[...], (((1,), (1,)), ((), ())),
                        preferred_element_type=jnp.float32)
    pad = jnp.zeros((x_ref.shape[0], _ROW - _CR - _CHANNELS - _CHUNK), jnp.float32)
    out_ref[...] = jnp.concatenate([x_ref[...], y_ref[...], t, pad], axis=1)


def _build_table(x_embed, y_embed, f_embed, fc1_w, fc1_b, fc2_w):
    NL = x_embed.shape[0]
    BLK = 1024
    grid = (NL // BLK,)
    return pl.pallas_call(
        _embed_body,
        grid=grid,
        in_specs=[
            pl.BlockSpec((BLK, _CR), lambda i: (i, 0)),
            pl.BlockSpec((BLK, _CHANNELS), lambda i: (i, 0)),
            pl.BlockSpec((BLK, _CHANNELS), lambda i: (i, 0)),
            pl.BlockSpec((_CHUNK, _CHANNELS), lambda i: (0, 0)),
            pl.BlockSpec((1, _CHUNK), lambda i: (0, 0)),
            pl.BlockSpec((_CHUNK, _CHUNK), lambda i: (0, 0)),
        ],
        out_specs=pl.BlockSpec((BLK, _ROW), lambda i: (i, 0)),
        out_shape=jax.ShapeDtypeStruct((NL, _ROW), jnp.float32),
    )(x_embed, y_embed, f_embed, fc1_w, fc1_b.reshape(1, -1), fc2_w)


def _sc_gather(table, gidx3d, D):
    B = _NW * gidx3d.shape[1] * _SCCHUNK
    per_w = B // _NW
    n_ch = per_w // _SCCHUNK
    mesh = plsc.VectorSubcoreMesh(core_axis_name="c", subcore_axis_name="s")

    @functools.partial(
        pl.kernel, mesh=mesh,
        out_type=jax.ShapeDtypeStruct((B, D), jnp.float32),
        scratch_types=[
            pltpu.VMEM((n_ch, _SCCHUNK), jnp.int32),
            pltpu.VMEM((_SCCHUNK, D), jnp.float32),
            pltpu.SemaphoreType.DMA,
        ],
    )
    def k(table_hbm, idx_hbm, out_hbm, idx_v, rows_v, sem):
        wid = lax.axis_index("s") * 2 + lax.axis_index("c")
        pltpu.sync_copy(idx_hbm.at[wid], idx_v)

        def body(j, _):
            pltpu.async_copy(table_hbm.at[idx_v.at[j]], rows_v, sem).wait()
            base = wid * per_w + j * _SCCHUNK
            pltpu.sync_copy(rows_v, out_hbm.at[pl.ds(base, _SCCHUNK)])
            return 0

        lax.fori_loop(0, n_ch, body, 0)

    return k(table, gidx3d)


def _sc_scatter(rows, gdst3d, D):
    B = _NW * gdst3d.shape[1] * _SCCHUNK
    per_w = B // _NW
    n_ch = per_w // _SCCHUNK
    mesh = plsc.VectorSubcoreMesh(core_axis_name="c", subcore_axis_name="s")

    @functools.partial(
        pl.kernel, mesh=mesh,
        out_type=jax.ShapeDtypeStruct((B, D), jnp.float32),
        scratch_types=[
            pltpu.VMEM((n_ch, _SCCHUNK), jnp.int32),
            pltpu.VMEM((_SCCHUNK, D), jnp.float32),
            pltpu.SemaphoreType.DMA,
        ],
    )
    def k(rows_hbm, idx_hbm, out_hbm, idx_v, rows_v, sem):
        wid = lax.axis_index("s") * 2 + lax.axis_index("c")
        pltpu.sync_copy(idx_hbm.at[wid], idx_v)

        def body(j, _):
            base = wid * per_w + j * _SCCHUNK
            pltpu.sync_copy(rows_hbm.at[pl.ds(base, _SCCHUNK)], rows_v)
            pltpu.async_copy(rows_v, out_hbm.at[idx_v.at[j]], sem).wait()
            return 0

        lax.fori_loop(0, n_ch, body, 0)

    return k(rows, gdst3d)


def _attn_body(rows_ref, fc2_b, out_ref):
    K = rows_ref.shape[1] // _CHUNK
    eye = jnp.eye(_CHUNK, dtype=jnp.float32)
    zpad = jnp.zeros((_CHUNK, _OROW - _CHANNELS - 1), dtype=jnp.float32)

    def chunk(start):
        return rows_ref[0, pl.ds(start, _CHUNK), :]

    def body(k, _):
        cur = chunk(k * _CHUNK)
        prv = chunk(lax.rem(k + K - 1, K) * _CHUNK)
        nxt = chunk(lax.rem(k + 1, K) * _CHUNK)
        xq = cur[:, :_CR]

        def nrm(t):
            x = t[:, :_CR]
            n = jnp.sqrt(jnp.sum(x * x, axis=-1, keepdims=True))
            return x / jnp.maximum(n, 5e-05)

        xm = jnp.concatenate([nrm(cur), nrm(prv), nrm(nxt)], axis=0)
        yc = jnp.concatenate([cur[:, _CR:_CR + _CHANNELS],
                              prv[:, _CR:_CR + _CHANNELS],
                              nxt[:, _CR:_CR + _CHANNELS]], axis=0)
        t0 = _CR + _CHANNELS
        tc = jnp.concatenate([cur[:, t0:t0 + _CHUNK],
                              prv[:, t0:t0 + _CHUNK],
                              nxt[:, t0:t0 + _CHUNK]], axis=0)

        raw_t = lax.dot_general(xm, xq, (((1,), (1,)), ((), ())),
                                preferred_element_type=jnp.float32) + tc + fc2_b[...]
        m = jnp.max(raw_t, axis=0, keepdims=True)
        e = jnp.exp(raw_t - m)
        s = jnp.sum(e, axis=0, keepdims=True)
        score = e / s
        ret = lax.dot_general(score, yc, (((0,), (0,)), ((), ())),
                              preferred_element_type=jnp.float32)
        bs = m + jnp.log(s)
        bcol = lax.dot_general(eye, bs, (((1,), (1,)), ((), ())),
                               preferred_element_type=jnp.float32)
        out_ref[0, pl.ds(k * _CHUNK, _CHUNK), :] = jnp.concatenate(
            [ret, bcol, zpad], axis=1)
        return 0

    lax.fori_loop(0, K, body, 0)


def _bucket_attention(rows_sorted, fc2_b, G, LH):
    return pl.pallas_call(
        _attn_body,
        grid=(G,),
        in_specs=[
            pl.BlockSpec((1, LH, _ROW), lambda h: (h, 0, 0)),
            pl.BlockSpec((1, _CHUNK), lambda h: (0, 0)),
        ],
        out_specs=pl.BlockSpec((1, LH, _OROW), lambda h: (h, 0, 0)),
        out_shape=jax.ShapeDtypeStruct((G, LH, _OROW), jnp.float32),
    )(rows_sorted, fc2_b.reshape(1, -1))


def _conv2d_relu(x, w, b):
    y = lax.conv_general_dilated(x, w, window_strides=(1, 1), padding='SAME',
                                 dimension_numbers=('NCHW', 'OIHW', 'NCHW'))
    return jax.nn.relu(y + b[None, :, None, None])


def kernel(input, cm_w, cm_b, ca_w, ca_b, cf_w, cf_b, fc1_w, fc1_b, fc2_w, fc2_b, random_rotations):
    x_nchw = jnp.transpose(input, (0, 3, 1, 2))
    N, _, H, W = x_nchw.shape
    L = H * W
    x_embed = _conv2d_relu(x_nchw, cm_w, cm_b).reshape(N, -1, L).transpose(0, 2, 1)
    y_embed = _conv2d_relu(x_nchw, ca_w, ca_b).reshape(N, -1, L).transpose(0, 2, 1)
    fc_embed = _conv2d_relu(x_nchw, cf_w, cf_b).reshape(N, -1, L).transpose(0, 2, 1)
    C = x_embed.shape[-1]
    hb = min(L // _CHUNK + (L // _CHUNK) % 2, 128)

    rot = random_rotations.reshape(C, _N_HASHES, hb)
    rotated = jnp.einsum('btf,fhi->bhti', x_embed, rot)
    hash_codes = jnp.argmax(rotated, axis=-1)
    offsets = (jnp.arange(_N_HASHES) * hb).reshape(1, -1, 1)
    hash_codes = (hash_codes + offsets).reshape(N, -1)
    indices = jnp.argsort(hash_codes, axis=-1)

    table = _build_table(x_embed.reshape(N * L, C),
                         y_embed.reshape(N * L, _CHANNELS),
                         fc_embed.reshape(N * L, _CHANNELS),
                         fc1_w, fc1_b, fc2_w)

    HL = _N_HASHES * L
    gidx = (indices % L + (jnp.arange(N) * L)[:, None]).astype(jnp.int32)
    rows_sorted = _sc_gather(table, gidx.reshape(_NW, -1, _SCCHUNK), _ROW)

    G = N * _N_HASHES
    LH = L
    out80 = _bucket_attention(rows_sorted.reshape(G, LH, _ROW), fc2_b, G, LH)

    gdst = (indices + (jnp.arange(N) * HL)[:, None]).astype(jnp.int32)
    unsorted = _sc_scatter(out80.reshape(N * HL, _OROW), gdst.reshape(_NW, -1, _SCCHUNK), _OROW)

    ret = unsorted[:, :_CHANNELS].reshape(N, _N_HASHES, L, _CHANNELS)
    bscore = unsorted[:, _CHANNELS].reshape(N, _N_HASHES, L, 1)
    probs = jax.nn.softmax(bscore, axis=1)
    ret = jnp.sum(ret * probs, axis=1).reshape(N, H, W, -1)
    return ret + input
